# Initial kernel scaffold; baseline (speedup 1.0000x reference)
import jax
import jax.numpy as jnp
from jax.experimental import pallas as pl


def kernel(x, ln_g, ln_b, Wattn, battn, Wg, W1, b1, W2, b2, Wn, bn):
    raise NotImplementedError("write your pallas kernel here")


# speedup vs baseline: 3.8697x; 3.8697x over previous
import jax
import jax.numpy as jnp
from jax.experimental import pallas as pl
from jax.experimental.pallas import tpu as pltpu

_MICRO = 2
_FBLK = 1024


def _gelu(v):
    return v * 0.5 * (1.0 + jax.lax.erf(v * 0.7071067811865476))


def _step(x_ref, ln_g_ref, ln_b_ref, wattn_ref, battn_ref, w1_ref, b1_ref,
          w2_ref, b2_ref, wn_ref, bn_ref, out_ref, xa_s, oacc_s):
    f = pl.program_id(1)
    nf = pl.num_programs(1)
    t, d = xa_s.shape

    @pl.when(f == 0)
    def _pre():
        c = x_ref[...].reshape(t, d)
        mu = jnp.mean(c, axis=-1, keepdims=True)
        var = jnp.mean((c - mu) ** 2, axis=-1, keepdims=True)
        xn = (c - mu) / jnp.sqrt(var + 1e-5) * ln_g_ref[...] + ln_b_ref[...]
        xa = jnp.dot(xn.astype(jnp.bfloat16), wattn_ref[...].astype(jnp.bfloat16),
                     preferred_element_type=jnp.float32) + battn_ref[...] + c
        xa_s[...] = xa.astype(jnp.bfloat16)

    h = jnp.dot(xa_s[...], w1_ref[0].astype(jnp.bfloat16),
                preferred_element_type=jnp.float32)
    h = _gelu(h + b1_ref[0])
    contrib = jnp.dot(h.astype(jnp.bfloat16), w2_ref[0].astype(jnp.bfloat16),
                      preferred_element_type=jnp.float32)

    @pl.when(f == 0)
    def _init():
        oacc_s[...] = contrib

    @pl.when(f > 0)
    def _acc():
        oacc_s[...] += contrib

    @pl.when(f == nf - 1)
    def _post():
        o = oacc_s[...] + b2_ref[0]
        y = jnp.dot(o.astype(jnp.bfloat16), wn_ref[...].astype(jnp.bfloat16),
                    preferred_element_type=jnp.float32) + bn_ref[...]
        out_ref[...] = _gelu(y).reshape(_MICRO, 1, t // _MICRO, d)


def kernel(x, ln_g, ln_b, Wattn, battn, Wg, W1, b1, W2, b2, Wn, bn):
    B, S, D = x.shape
    E, _, H = W1.shape
    F = H // _FBLK
    tpb = (B * S) // (_MICRO * E)
    t = _MICRO * tpb

    xr = x.reshape(_MICRO, E, tpb, D)

    out = pl.pallas_call(
        _step,
        grid=(E, F),
        in_specs=[
            pl.BlockSpec((_MICRO, 1, tpb, D), lambda e, f: (0, e, 0, 0)),
            pl.BlockSpec((1, D), lambda e, f: (0, 0)),
            pl.BlockSpec((1, D), lambda e, f: (0, 0)),
            pl.BlockSpec((D, D), lambda e, f: (0, 0)),
            pl.BlockSpec((1, D), lambda e, f: (0, 0)),
            pl.BlockSpec((1, D, _FBLK), lambda e, f: (e, 0, f)),
            pl.BlockSpec((1, 1, _FBLK), lambda e, f: (e, 0, f)),
            pl.BlockSpec((1, _FBLK, D), lambda e, f: (e, f, 0)),
            pl.BlockSpec((1, 1, D), lambda e, f: (e, 0, 0)),
            pl.BlockSpec((D, D), lambda e, f: (0, 0)),
            pl.BlockSpec((1, D), lambda e, f: (0, 0)),
        ],
        out_specs=pl.BlockSpec((_MICRO, 1, tpb, D), lambda e, f: (0, e, 0, 0)),
        out_shape=jax.ShapeDtypeStruct((_MICRO, E, tpb, D), jnp.float32),
        scratch_shapes=[
            pltpu.VMEM((t, D), jnp.bfloat16),
            pltpu.VMEM((t, D), jnp.float32),
        ],
        compiler_params=pltpu.CompilerParams(
            dimension_semantics=("arbitrary", "arbitrary"),
        ),
    )(xr, ln_g.reshape(1, D), ln_b.reshape(1, D), Wattn, battn.reshape(1, D),
      W1, b1.reshape(E, 1, H), W2, b2.reshape(E, 1, D), Wn, bn.reshape(1, D))
    return out.reshape(B, S, D)


# speedup vs baseline: 3.8704x; 1.0002x over previous
import jax
import jax.numpy as jnp
from jax.experimental import pallas as pl
from jax.experimental.pallas import tpu as pltpu

_MICRO = 2
_FBLK = 1024


def _gelu(v):
    return v * 0.5 * (1.0 + jax.lax.erf(v * 0.7071067811865476))


def _step(x_ref, ln_g_ref, ln_b_ref, wattn_ref, battn_ref, w1_ref, b1_ref,
          w2_ref, b2_ref, wn_ref, bn_ref, out_ref, xa_s, oacc_s):
    f = pl.program_id(1)
    nf = pl.num_programs(1)
    t, d = xa_s.shape

    @pl.when(f == 0)
    def _pre():
        c = x_ref[...].reshape(t, d)
        mu = jnp.mean(c, axis=-1, keepdims=True)
        var = jnp.mean((c - mu) ** 2, axis=-1, keepdims=True)
        xn = (c - mu) / jnp.sqrt(var + 1e-5) * ln_g_ref[...] + ln_b_ref[...]
        xa = jnp.dot(xn.astype(jnp.bfloat16), wattn_ref[...].astype(jnp.bfloat16),
                     preferred_element_type=jnp.float32) + battn_ref[...] + c
        xa_s[...] = xa.astype(jnp.bfloat16)

    h = jnp.dot(xa_s[...], w1_ref[0].astype(jnp.bfloat16),
                preferred_element_type=jnp.float32)
    h = _gelu(h + b1_ref[0])
    contrib = jnp.dot(h.astype(jnp.bfloat16), w2_ref[0].astype(jnp.bfloat16),
                      preferred_element_type=jnp.float32)

    @pl.when(f == 0)
    def _init():
        oacc_s[...] = contrib

    @pl.when(f > 0)
    def _acc():
        oacc_s[...] += contrib

    @pl.when(f == nf - 1)
    def _post():
        o = oacc_s[...] + b2_ref[0]
        y = jnp.dot(o.astype(jnp.bfloat16), wn_ref[...].astype(jnp.bfloat16),
                    preferred_element_type=jnp.float32) + bn_ref[...]
        out_ref[...] = _gelu(y).reshape(_MICRO, 1, t // _MICRO, d)


def kernel(x, ln_g, ln_b, Wattn, battn, Wg, W1, b1, W2, b2, Wn, bn):
    B, S, D = x.shape
    E, _, H = W1.shape
    F = H // _FBLK
    tpb = (B * S) // (_MICRO * E)
    t = _MICRO * tpb

    xr = x.reshape(_MICRO, E, tpb, D)

    out = pl.pallas_call(
        _step,
        grid=(E, F),
        in_specs=[
            pl.BlockSpec((_MICRO, 1, tpb, D), lambda e, f: (0, e, 0, 0)),
            pl.BlockSpec((1, D), lambda e, f: (0, 0)),
            pl.BlockSpec((1, D), lambda e, f: (0, 0)),
            pl.BlockSpec((D, D), lambda e, f: (0, 0)),
            pl.BlockSpec((1, D), lambda e, f: (0, 0)),
            pl.BlockSpec((1, D, _FBLK), lambda e, f: (e, 0, f)),
            pl.BlockSpec((1, 1, _FBLK), lambda e, f: (e, 0, f)),
            pl.BlockSpec((1, _FBLK, D), lambda e, f: (e, f, 0)),
            pl.BlockSpec((1, 1, D), lambda e, f: (e, 0, 0)),
            pl.BlockSpec((D, D), lambda e, f: (0, 0)),
            pl.BlockSpec((1, D), lambda e, f: (0, 0)),
        ],
        out_specs=pl.BlockSpec((_MICRO, 1, tpb, D), lambda e, f: (0, e, 0, 0)),
        out_shape=jax.ShapeDtypeStruct((_MICRO, E, tpb, D), jnp.float32),
        scratch_shapes=[
            pltpu.VMEM((t, D), jnp.bfloat16),
            pltpu.VMEM((t, D), jnp.float32),
        ],
        compiler_params=pltpu.CompilerParams(
            dimension_semantics=("parallel", "arbitrary"),
        ),
    )(xr, ln_g.reshape(1, D), ln_b.reshape(1, D), Wattn, battn.reshape(1, D),
      W1, b1.reshape(E, 1, H), W2, b2.reshape(E, 1, D), Wn, bn.reshape(1, D))
    return out.reshape(B, S, D)
